# baseline (speedup 1.0000x reference)
import dataclasses
import functools

import jax
import jax.numpy as jnp
from jax import lax
from jax.experimental import pallas as pl
from jax.experimental.pallas import tpu as pltpu
from jax.experimental.pallas import tpu_sc as plsc

N = 10000
E = 320000
D_IN = 128
HID = 128
NCL = 16

NC = 2
NS = 16
LANES = 16
NW = NC * NS

CHUNK = 128
EP = 327680
EPW = EP // NW
NCHUNKS = EPW // CHUNK
NP = 10240
NPAD = NP - N
RPS = NP // NS
ZROWS = 128


def _mesh():
    return plsc.VectorSubcoreMesh(core_axis_name="c", subcore_axis_name="s")


def _sc_compiler_params():
    cp = pltpu.CompilerParams()
    if "needs_layout_passes" in pltpu.CompilerParams.__dataclass_fields__:
        cp = dataclasses.replace(cp, needs_layout_passes=False)
    return cp


@functools.partial(
    pl.kernel,
    out_type=jax.ShapeDtypeStruct((NW, NP), jnp.float32),
    mesh=_mesh(),
    compiler_params=_sc_compiler_params(),
    scratch_types=[
        pltpu.VMEM((NP,), jnp.float32),
        pltpu.VMEM((NCHUNKS, CHUNK), jnp.int32),
    ],
)
def _deg_kernel(edges_hbm, out_hbm, hist, didx):
    c = lax.axis_index("c")
    s = lax.axis_index("s")
    w = c * NS + s

    @pl.loop(0, NP, step=LANES)
    def _(i):
        hist[pl.ds(i, LANES)] = jnp.zeros((LANES,), jnp.float32)

    pltpu.sync_copy(edges_hbm.at[1, pl.ds(w * NCHUNKS, NCHUNKS)], didx)

    ones = jnp.ones((LANES,), jnp.float32)

    @pl.loop(0, NCHUNKS)
    def _(r):
        @pl.loop(0, CHUNK, step=LANES)
        def _(k):
            plsc.addupdate_scatter(hist, [didx[r, pl.ds(k, LANES)]], ones)

    pltpu.sync_copy(hist, out_hbm.at[w])


def _make_agg(D):
    PHASES = 2
    WCH = NCHUNKS // PHASES

    @functools.partial(
        pl.kernel,
        out_type=jax.ShapeDtypeStruct((NC, NP, D), jnp.float32),
        mesh=_mesh(),
        scratch_types=[
            pltpu.VMEM_SHARED((NP, D), jnp.float32),
            pltpu.VMEM((WCH, CHUNK), jnp.int32),
            pltpu.VMEM((WCH, CHUNK), jnp.int32),
            pltpu.VMEM((CHUNK, D), jnp.float32),
            pltpu.VMEM((CHUNK, D), jnp.float32),
            pltpu.SemaphoreType.DMA,
            pltpu.SemaphoreType.DMA,
            pltpu.SemaphoreType.DMA,
            pltpu.SemaphoreType.DMA,
        ],
    )
    def agg_kernel(edges_hbm, z_hbm, g_hbm, out_hbm,
                   acc, sidx, didx, rows0, rows1, sem0, sem1, zsem, _unused):
        c = lax.axis_index("c")
        s = lax.axis_index("s")
        w = c * NS + s
        rows = (rows0, rows1)
        sems = (sem0, sem1)

        def start(j, b):
            pltpu.async_copy(g_hbm.at[sidx.at[j]], rows[b], sems[b])

        def wait(j, b):
            pltpu.make_async_copy(g_hbm.at[sidx.at[j]], rows[b], sems[b]).wait()

        def scat(j, b):
            pltpu.sync_copy(rows[b], acc.at[didx.at[j]], add=True)

        @pl.loop(0, RPS, step=ZROWS)
        def _(r):
            pltpu.async_copy(z_hbm, acc.at[pl.ds(s * RPS + r, ZROWS)], zsem)

        pltpu.sync_copy(edges_hbm.at[0, pl.ds(w * NCHUNKS, WCH)], sidx)
        pltpu.sync_copy(edges_hbm.at[1, pl.ds(w * NCHUNKS, WCH)], didx)
        start(0, 0)
        start(1, 1)

        @pl.loop(0, RPS, step=ZROWS)
        def _(r):
            pltpu.make_async_copy(z_hbm, acc.at[pl.ds(s * RPS + r, ZROWS)], zsem).wait()

        plsc.subcore_barrier()

        @pl.loop(0, PHASES)
        def _(p):
            base = w * NCHUNKS + p * WCH

            @pl.when(p > 0)
            def _():
                pltpu.sync_copy(edges_hbm.at[0, pl.ds(base, WCH)], sidx)
                pltpu.sync_copy(edges_hbm.at[1, pl.ds(base, WCH)], didx)
                start(0, 0)
                start(1, 1)

            @pl.loop(0, WCH - 2, step=2)
            def _(j):
                wait(j, 0)
                scat(j, 0)
                start(j + 2, 0)
                wait(j + 1, 1)
                scat(j + 1, 1)
                start(j + 3, 1)

            wait(WCH - 2, 0)
            scat(WCH - 2, 0)
            wait(WCH - 1, 1)
            scat(WCH - 1, 1)

        plsc.subcore_barrier()
        pltpu.sync_copy(acc.at[pl.ds(s * RPS, RPS)], out_hbm.at[c, pl.ds(s * RPS, RPS)])

    return agg_kernel


_agg128 = _make_agg(HID)


def _dis_from_hist(degh_ref):
    deg = jnp.sum(degh_ref[...], axis=0)[0:N] + 1.0
    return lax.rsqrt(deg)[:, None]


def _mm_body(x_ref, w_ref, o_ref):
    o_ref[...] = jnp.dot(x_ref[...], w_ref[...], preferred_element_type=jnp.float32)


def _tc_matmul(x, w):
    return pl.pallas_call(
        _mm_body,
        out_shape=jax.ShapeDtypeStruct((x.shape[0], w.shape[1]), jnp.float32),
    )(x, w)


def _g1_body(h_ref, degh_ref, g_ref):
    g_ref[...] = h_ref[...] * _dis_from_hist(degh_ref)


def _tc_g1(h, degh):
    return pl.pallas_call(
        _g1_body,
        out_shape=jax.ShapeDtypeStruct(h.shape, jnp.float32),
    )(h, degh)


def _combine_body(aggp_ref, h_ref, degh_ref, w_ref, b_ref, hn_ref, gn_ref):
    dis = _dis_from_hist(degh_ref)
    agg = aggp_ref[0, 0:N] + aggp_ref[1, 0:N]
    z = jnp.maximum(dis * agg + (dis * dis) * h_ref[...] + b_ref[...], 0.0)
    hn = jnp.dot(z, w_ref[...], preferred_element_type=jnp.float32)
    hn_ref[...] = hn
    gn_ref[...] = hn * dis


def _tc_combine(aggp, h, degh, w, b):
    d_out = w.shape[1]
    return pl.pallas_call(
        _combine_body,
        out_shape=[
            jax.ShapeDtypeStruct((N, d_out), jnp.float32),
            jax.ShapeDtypeStruct((N, d_out), jnp.float32),
        ],
    )(aggp, h, degh, w, b)


def _combine3_body(aggp_ref, h_ref, degh_ref, b_ref, z_ref, y_ref):
    dis = _dis_from_hist(degh_ref)
    agg = aggp_ref[0, 0:N] + aggp_ref[1, 0:N]
    z = jnp.maximum(dis * agg + (dis * dis) * h_ref[...] + b_ref[...], 0.0)
    z_ref[...] = z
    y_ref[...] = z * dis


def _tc_combine3(aggp, h, degh, b):
    return pl.pallas_call(
        _combine3_body,
        out_shape=[
            jax.ShapeDtypeStruct((N, HID), jnp.float32),
            jax.ShapeDtypeStruct((N, HID), jnp.float32),
        ],
    )(aggp, h, degh, b)


def _final_body(aggp_ref, z_ref, degh_ref, w_ref, b_ref, o_ref):
    dis = _dis_from_hist(degh_ref)
    agg = aggp_ref[0, 0:N] + aggp_ref[1, 0:N]
    t = dis * agg + (dis * dis) * z_ref[...]
    o_ref[...] = jnp.dot(t, w_ref[...], preferred_element_type=jnp.float32) + b_ref[...]


def _tc_final(aggp, z2, degh, w, b):
    return pl.pallas_call(
        _final_body,
        out_shape=jax.ShapeDtypeStruct((N, NCL), jnp.float32),
    )(aggp, z2, degh, w, b)


def kernel(x, edge_index, W1, b1, W2, b2, W3, b3):
    npad_e = EP - E
    pad_src = jnp.zeros((1, npad_e), jnp.int32)
    pad_dst = (N + jnp.arange(npad_e, dtype=jnp.int32) % NPAD)[None, :]
    edges = jnp.concatenate(
        [edge_index, jnp.concatenate([pad_src, pad_dst], axis=0)], axis=1
    ).reshape(2, EP // CHUNK, CHUNK)
    z128 = jnp.zeros((ZROWS, HID), jnp.float32)

    degh = _deg_kernel(edges)
    h1 = _tc_matmul(x, W1)
    g1 = _tc_g1(h1, degh)
    agg1 = _agg128(edges, z128, g1)
    h2, g2 = _tc_combine(agg1, h1, degh, W2, b1)
    agg2 = _agg128(edges, z128, g2)
    z2, y2 = _tc_combine3(agg2, h2, degh, b2)
    agg3 = _agg128(edges, z128, y2)
    return _tc_final(agg3, z2, degh, W3, b3)


# speedup vs baseline: 3.2169x; 3.2169x over previous
import dataclasses
import functools

import jax
import jax.numpy as jnp
from jax import lax
from jax.experimental import pallas as pl
from jax.experimental.pallas import tpu as pltpu
from jax.experimental.pallas import tpu_sc as plsc

N = 10000
E = 320000
D_IN = 128
HID = 128
NCL = 16

NC = 2
NS = 16
LANES = 16
NW = NC * NS

CHUNK = 128
EP = 327680
EPW = EP // NW
NCHUNKS = EPW // CHUNK
NP = 10240
NPAD = NP - N
RPS = NP // NS
ZROWS = 128


def _mesh():
    return plsc.VectorSubcoreMesh(core_axis_name="c", subcore_axis_name="s")


def _sc_compiler_params():
    cp = pltpu.CompilerParams()
    if "needs_layout_passes" in pltpu.CompilerParams.__dataclass_fields__:
        cp = dataclasses.replace(cp, needs_layout_passes=False)
    return cp


@functools.partial(
    pl.kernel,
    out_type=jax.ShapeDtypeStruct((NW, NP), jnp.float32),
    mesh=_mesh(),
    compiler_params=_sc_compiler_params(),
    scratch_types=[
        pltpu.VMEM((NP,), jnp.float32),
        pltpu.VMEM((NCHUNKS, CHUNK), jnp.int32),
    ],
)
def _deg_kernel(edges_hbm, out_hbm, hist, didx):
    c = lax.axis_index("c")
    s = lax.axis_index("s")
    w = c * NS + s

    @pl.loop(0, NP, step=LANES)
    def _(i):
        hist[pl.ds(i, LANES)] = jnp.zeros((LANES,), jnp.float32)

    pltpu.sync_copy(edges_hbm.at[1, pl.ds(w * NCHUNKS, NCHUNKS)], didx)

    ones = jnp.ones((LANES,), jnp.float32)

    @pl.loop(0, NCHUNKS)
    def _(r):
        @pl.loop(0, CHUNK, step=LANES)
        def _(k):
            plsc.addupdate_scatter(hist, [didx[r, pl.ds(k, LANES)]], ones)

    pltpu.sync_copy(hist, out_hbm.at[w])


def _make_agg(D):
    PHASES = 2
    WCH = NCHUNKS // PHASES

    @functools.partial(
        pl.kernel,
        out_type=jax.ShapeDtypeStruct((NC, NP, D), jnp.float32),
        mesh=_mesh(),
        scratch_types=[
            pltpu.VMEM_SHARED((NP, D), jnp.float32),
            pltpu.VMEM((WCH, CHUNK), jnp.int32),
            pltpu.VMEM((WCH, CHUNK), jnp.int32),
            pltpu.VMEM((CHUNK, D), jnp.float32),
            pltpu.VMEM((CHUNK, D), jnp.float32),
            pltpu.SemaphoreType.DMA,
            pltpu.SemaphoreType.DMA,
            pltpu.SemaphoreType.DMA,
            pltpu.SemaphoreType.DMA,
        ],
    )
    def agg_kernel(edges_hbm, z_hbm, g_hbm, out_hbm,
                   acc, sidx, didx, rows0, rows1, sem0, sem1, zsem, _unused):
        c = lax.axis_index("c")
        s = lax.axis_index("s")
        w = c * NS + s
        rows = (rows0, rows1)
        sems = (sem0, sem1)

        def start(j, b):
            pltpu.async_copy(g_hbm.at[sidx.at[j]], rows[b], sems[b])

        def wait(j, b):
            pltpu.make_async_copy(g_hbm.at[sidx.at[j]], rows[b], sems[b]).wait()

        def scat(j, b):
            pltpu.sync_copy(rows[b], acc.at[didx.at[j]], add=True)

        @pl.loop(0, RPS, step=ZROWS)
        def _(r):
            pltpu.async_copy(z_hbm, acc.at[pl.ds(s * RPS + r, ZROWS)], zsem)

        pltpu.sync_copy(edges_hbm.at[0, pl.ds(w * NCHUNKS, WCH)], sidx)
        pltpu.sync_copy(edges_hbm.at[1, pl.ds(w * NCHUNKS, WCH)], didx)
        start(0, 0)
        start(1, 1)

        @pl.loop(0, RPS, step=ZROWS)
        def _(r):
            pltpu.make_async_copy(z_hbm, acc.at[pl.ds(s * RPS + r, ZROWS)], zsem).wait()

        plsc.subcore_barrier()

        @pl.loop(0, PHASES)
        def _(p):
            base = w * NCHUNKS + p * WCH

            @pl.when(p > 0)
            def _():
                pltpu.sync_copy(edges_hbm.at[0, pl.ds(base, WCH)], sidx)
                pltpu.sync_copy(edges_hbm.at[1, pl.ds(base, WCH)], didx)
                start(0, 0)
                start(1, 1)

            @pl.loop(0, WCH - 2, step=2)
            def _(j):
                wait(j, 0)
                scat(j, 0)
                start(j + 2, 0)
                wait(j + 1, 1)
                scat(j + 1, 1)
                start(j + 3, 1)

            wait(WCH - 2, 0)
            scat(WCH - 2, 0)
            wait(WCH - 1, 1)
            scat(WCH - 1, 1)

        plsc.subcore_barrier()
        pltpu.sync_copy(acc.at[pl.ds(s * RPS, RPS)], out_hbm.at[c, pl.ds(s * RPS, RPS)])

    return agg_kernel


_agg128 = _make_agg(HID)


def _dis_from_hist(degh_ref):
    deg = jnp.sum(degh_ref[...], axis=0)[0:N] + 1.0
    return lax.rsqrt(deg)[:, None]


def _mm_body(x_ref, w_ref, o_ref):
    o_ref[...] = jnp.dot(x_ref[...], w_ref[...], preferred_element_type=jnp.float32)


def _tc_matmul(x, w):
    return pl.pallas_call(
        _mm_body,
        out_shape=jax.ShapeDtypeStruct((x.shape[0], w.shape[1]), jnp.float32),
    )(x, w)


def _g1_body(h_ref, degh_ref, g_ref):
    g_ref[...] = h_ref[...] * _dis_from_hist(degh_ref)


def _tc_g1(h, degh):
    return pl.pallas_call(
        _g1_body,
        out_shape=jax.ShapeDtypeStruct(h.shape, jnp.float32),
    )(h, degh)


def _combine_body(aggp_ref, h_ref, degh_ref, w_ref, b_ref, hn_ref, gn_ref):
    dis = _dis_from_hist(degh_ref)
    agg = aggp_ref[0, 0:N] + aggp_ref[1, 0:N]
    z = jnp.maximum(dis * agg + (dis * dis) * h_ref[...] + b_ref[...], 0.0)
    hn = jnp.dot(z, w_ref[...], preferred_element_type=jnp.float32)
    hn_ref[...] = hn
    gn_ref[...] = hn * dis


def _tc_combine(aggp, h, degh, w, b):
    d_out = w.shape[1]
    return pl.pallas_call(
        _combine_body,
        out_shape=[
            jax.ShapeDtypeStruct((N, d_out), jnp.float32),
            jax.ShapeDtypeStruct((N, d_out), jnp.float32),
        ],
    )(aggp, h, degh, w, b)


def _combine3_body(aggp_ref, h_ref, degh_ref, b_ref, z_ref, y_ref):
    dis = _dis_from_hist(degh_ref)
    agg = aggp_ref[0, 0:N] + aggp_ref[1, 0:N]
    z = jnp.maximum(dis * agg + (dis * dis) * h_ref[...] + b_ref[...], 0.0)
    z_ref[...] = z
    y_ref[...] = z * dis


def _tc_combine3(aggp, h, degh, b):
    return pl.pallas_call(
        _combine3_body,
        out_shape=[
            jax.ShapeDtypeStruct((N, HID), jnp.float32),
            jax.ShapeDtypeStruct((N, HID), jnp.float32),
        ],
    )(aggp, h, degh, b)


def _final_body(aggp_ref, z_ref, degh_ref, w_ref, b_ref, o_ref):
    dis = _dis_from_hist(degh_ref)
    agg = aggp_ref[0, 0:N] + aggp_ref[1, 0:N]
    t = dis * agg + (dis * dis) * z_ref[...]
    o_ref[...] = jnp.dot(t, w_ref[...], preferred_element_type=jnp.float32) + b_ref[...]


def _tc_final(aggp, z2, degh, w, b):
    return pl.pallas_call(
        _final_body,
        out_shape=jax.ShapeDtypeStruct((N, NCL), jnp.float32),
    )(aggp, z2, degh, w, b)


def kernel(x, edge_index, W1, b1, W2, b2, W3, b3):
    npad_e = EP - E
    pad_iota = jnp.arange(npad_e, dtype=jnp.int32)
    pad_src = (pad_iota * 37 % N)[None, :]
    pad_dst = (N + pad_iota % NPAD)[None, :]
    edges = jnp.concatenate(
        [edge_index, jnp.concatenate([pad_src, pad_dst], axis=0)], axis=1
    ).reshape(2, EP // CHUNK, CHUNK)
    z128 = jnp.zeros((ZROWS, HID), jnp.float32)

    degh = _deg_kernel(edges)
    h1 = _tc_matmul(x, W1)
    g1 = _tc_g1(h1, degh)
    agg1 = _agg128(edges, z128, g1)
    h2, g2 = _tc_combine(agg1, h1, degh, W2, b1)
    agg2 = _agg128(edges, z128, g2)
    z2, y2 = _tc_combine3(agg2, h2, degh, b2)
    agg3 = _agg128(edges, z128, y2)
    return _tc_final(agg3, z2, degh, W3, b3)
